# baseline (speedup 1.0000x reference)
import functools

import jax
import jax.numpy as jnp
from jax import lax
from jax.experimental import pallas as pl
from jax.experimental.pallas import tpu as pltpu
from jax.experimental.pallas import tpu_sc as plsc

_D = 1024
_S = 8192
_B = 4
_NC = 2
_NS = 16
_NW = _NC * _NS
_RPW = _S // _NW
_Q = 64
_NQ = _RPW // _Q
_CH = 8
_NPAIR = _Q // (2 * _CH)
_VL = 16


def _body(x_hbm, cls_hbm, pos_hbm, out_hbm,
          x0, x1, o0, o1, pq, posz, oz, sx0, sx1, so0, so1):
    cid = lax.axis_index("c")
    sid = lax.axis_index("s")
    wid = sid * _NC + cid
    base = wid * _RPW

    X = (x0, x1)
    O = (o0, o1)
    SX = (sx0, sx1)
    SO = (so0, so1)

    def q_body(q, carry_q):
        a = base + q * _Q
        pltpu.sync_copy(pos_hbm.at[pl.ds(a, _Q), :], pq)

        def b_body(b, carry_b):
            first = jnp.logical_and(wid == 0, q == 0)

            @pl.when(first)
            def _():
                pltpu.sync_copy(cls_hbm, x1.at[pl.ds(_CH - 1, 1), :])

            @pl.when(jnp.logical_not(first))
            def _():
                pltpu.sync_copy(x_hbm.at[b, pl.ds(a - _CH, _CH), :], x1)

            pltpu.async_copy(x_hbm.at[b, pl.ds(a, _CH), :], x0, sx0)

            def pair_body(j, carry_j):
                @pl.when(j > 0)
                def _():
                    pltpu.make_async_copy(
                        o0, out_hbm.at[b, pl.ds(a, _CH), :], so0).wait()
                    pltpu.make_async_copy(
                        o1, out_hbm.at[b, pl.ds(a, _CH), :], so1).wait()

                for k in range(2):
                    cc = 2 * j + k
                    p = k
                    row = a + cc * _CH
                    prow = cc * _CH
                    pltpu.make_async_copy(
                        x_hbm.at[b, pl.ds(row, _CH), :], X[p], SX[p]).wait()

                    @plsc.parallel_loop(0, _D, step=_VL, unroll=8)
                    def _(c):
                        ds = pl.ds(c, _VL)
                        O[p][0, ds] = X[1 - p][_CH - 1, ds] + pq[prow, ds]

                    nrow = a + jnp.minimum(cc + 1, 2 * _NPAIR - 1) * _CH
                    pltpu.async_copy(
                        x_hbm.at[b, pl.ds(nrow, _CH), :], X[1 - p], SX[1 - p])

                    @plsc.parallel_loop(1, _CH)
                    def _(r):
                        @plsc.parallel_loop(0, _D, step=_VL, unroll=8)
                        def _(c):
                            ds = pl.ds(c, _VL)
                            O[p][r, ds] = X[p][r - 1, ds] + pq[prow + r, ds]

                    pltpu.async_copy(
                        O[p], out_hbm.at[b, pl.ds(row, _CH), :], SO[p])
                return carry_j

            lax.fori_loop(0, _NPAIR, pair_body, 0)

            pltpu.make_async_copy(
                x_hbm.at[b, pl.ds(a, _CH), :], x0, sx0).wait()
            pltpu.make_async_copy(
                o0, out_hbm.at[b, pl.ds(a, _CH), :], so0).wait()
            pltpu.make_async_copy(
                o1, out_hbm.at[b, pl.ds(a, _CH), :], so1).wait()
            return carry_b

        lax.fori_loop(0, _B, b_body, 0)
        return carry_q

    lax.fori_loop(0, _NQ, q_body, 0)

    @pl.when(wid == _NW - 1)
    def _():
        pltpu.sync_copy(pos_hbm.at[pl.ds(_S, 1), :], posz)

        def z_body(b, carry):
            pltpu.sync_copy(x_hbm.at[b, pl.ds(_S - _CH, _CH), :], x0)

            @plsc.parallel_loop(0, _D, step=_VL, unroll=8)
            def _(c):
                ds = pl.ds(c, _VL)
                oz[0, ds] = x0[_CH - 1, ds] + posz[0, ds]

            pltpu.sync_copy(oz, out_hbm.at[b, pl.ds(_S, 1), :])
            return carry

        lax.fori_loop(0, _B, z_body, 0)


_pe_call = functools.partial(
    pl.kernel,
    out_type=jax.ShapeDtypeStruct((_B, _S + 1, _D), jnp.float32),
    mesh=plsc.VectorSubcoreMesh(core_axis_name="c", subcore_axis_name="s"),
    scratch_types=[
        pltpu.VMEM((_CH, _D), jnp.float32),
        pltpu.VMEM((_CH, _D), jnp.float32),
        pltpu.VMEM((_CH, _D), jnp.float32),
        pltpu.VMEM((_CH, _D), jnp.float32),
        pltpu.VMEM((_Q, _D), jnp.float32),
        pltpu.VMEM((1, _D), jnp.float32),
        pltpu.VMEM((1, _D), jnp.float32),
        pltpu.SemaphoreType.DMA,
        pltpu.SemaphoreType.DMA,
        pltpu.SemaphoreType.DMA,
        pltpu.SemaphoreType.DMA,
    ],
)(_body)


@jax.jit
def kernel(x, cls_token, pos_table):
    return _pe_call(x, cls_token.reshape(1, _D), pos_table)
